# baseline (speedup 1.0000x reference)
import jax
import jax.numpy as jnp
from jax import lax
from jax.experimental import pallas as pl
from jax.experimental.pallas import tpu as pltpu

B = 128
T = 4096
BR = 32
NBLK = B // BR


def _col(sm, base):
    return jnp.concatenate(
        [jnp.full((1, 1), sm[base + r], jnp.int32) for r in range(BR)], axis=0)


def _body(starts_sm, ends_sm, nf_sm, oidx_sm, e_ref, out_ref):
    g = pl.program_id(0)
    base = g * BR
    scol = _col(starts_sm, base)
    ecol = _col(ends_sm, base)
    fcol = _col(nf_sm, base)
    pos = lax.broadcasted_iota(jnp.int32, (BR, T), 1)
    in_win = (pos >= scol) & (pos < ecol)
    oracle = jnp.where(in_win, jnp.float32(1.0), jnp.float32(-99999.0))
    out_ref[...] = jnp.where(oidx_sm[0] < fcol, oracle, e_ref[...])


@jax.jit
def _tc_kernel(starts, ends, nf, oidx, e):
    grid_spec = pltpu.PrefetchScalarGridSpec(
        num_scalar_prefetch=4,
        grid=(NBLK,),
        in_specs=[pl.BlockSpec((BR, T), lambda i, *_: (i, 0))],
        out_specs=pl.BlockSpec((BR, T), lambda i, *_: (i, 0)),
    )
    return pl.pallas_call(
        _body,
        grid_spec=grid_spec,
        out_shape=jax.ShapeDtypeStruct((B, T), jnp.float32),
    )(starts, ends, nf, oidx, e)


def kernel(e, att_starts, att_ends, n_att_frames, output_index):
    oidx = jnp.asarray(output_index, jnp.int32).reshape(1)
    return _tc_kernel(att_starts.astype(jnp.int32), att_ends.astype(jnp.int32),
                      n_att_frames.astype(jnp.int32), oidx, e)


# speedup vs baseline: 1.0079x; 1.0079x over previous
import jax
import jax.numpy as jnp
from jax import lax
from jax.experimental import pallas as pl
from jax.experimental.pallas import tpu as pltpu

B = 128
T = 4096
BR = 64
NBLK = B // BR


def _col(sm, base):
    return jnp.concatenate(
        [jnp.full((1, 1), sm[base + r], jnp.int32) for r in range(BR)], axis=0)


def _body(starts_sm, ends_sm, nf_sm, oidx_sm, e_any, out_ref, sem, cnt):
    g = pl.program_id(0)
    base = g * BR
    oidx = oidx_sm[0]

    scol = _col(starts_sm, base)
    ecol = _col(ends_sm, base)
    pos = lax.broadcasted_iota(jnp.int32, (BR, T), 1)
    in_win = (pos >= scol) & (pos < ecol)
    out_ref[...] = jnp.where(in_win, jnp.float32(1.0), jnp.float32(-99999.0))

    cnt[0] = 0
    for r in range(BR):
        @pl.when(oidx >= nf_sm[base + r])
        def _():
            pltpu.make_async_copy(
                e_any.at[base + r], out_ref.at[r], sem).start()
            cnt[0] += 1

    def drain(_, carry):
        pltpu.make_async_copy(e_any.at[0], out_ref.at[0], sem).wait()
        return carry

    lax.fori_loop(0, cnt[0], drain, 0)


@jax.jit
def _tc_kernel(starts, ends, nf, oidx, e):
    grid_spec = pltpu.PrefetchScalarGridSpec(
        num_scalar_prefetch=4,
        grid=(NBLK,),
        in_specs=[pl.BlockSpec(memory_space=pl.ANY)],
        out_specs=pl.BlockSpec((BR, T), lambda i, *_: (i, 0)),
        scratch_shapes=[pltpu.SemaphoreType.DMA,
                        pltpu.SMEM((1,), jnp.int32)],
    )
    return pl.pallas_call(
        _body,
        grid_spec=grid_spec,
        out_shape=jax.ShapeDtypeStruct((B, T), jnp.float32),
    )(starts, ends, nf, oidx, e)


def kernel(e, att_starts, att_ends, n_att_frames, output_index):
    oidx = jnp.asarray(output_index, jnp.int32).reshape(1)
    return _tc_kernel(att_starts.astype(jnp.int32), att_ends.astype(jnp.int32),
                      n_att_frames.astype(jnp.int32), oidx, e)


# speedup vs baseline: 1.0582x; 1.0498x over previous
import jax
import jax.numpy as jnp
from jax import lax
from jax.experimental import pallas as pl
from jax.experimental.pallas import tpu as pltpu

B = 128
T = 4096
BR = 64
NBLK = B // BR


def _col(sm, base):
    return jnp.concatenate(
        [jnp.full((1, 1), sm[base + r], jnp.int32) for r in range(BR)], axis=0)


def _body(starts_sm, ends_sm, nf_sm, oidx_sm, e_any, out_ref, sem):
    g = pl.program_id(0)
    base = g * BR
    oidx = oidx_sm[0]

    scol = _col(starts_sm, base)
    ecol = _col(ends_sm, base)
    pos = lax.broadcasted_iota(jnp.int32, (BR, T), 1)
    in_win = (pos >= scol) & (pos < ecol)
    out_ref[...] = jnp.where(in_win, jnp.float32(1.0), jnp.float32(-99999.0))

    for r in range(BR):
        @pl.when(oidx >= nf_sm[base + r])
        def _():
            pltpu.make_async_copy(
                e_any.at[base + r], out_ref.at[r], sem).start()

    k = lax.fori_loop(
        0, BR,
        lambda r, acc: acc + (oidx >= nf_sm[base + r]).astype(jnp.int32),
        jnp.int32(0))

    def drain(_, carry):
        pltpu.make_async_copy(e_any.at[0], out_ref.at[0], sem).wait()
        return carry

    lax.fori_loop(0, k, drain, 0)


@jax.jit
def _tc_kernel(starts, ends, nf, oidx, e):
    grid_spec = pltpu.PrefetchScalarGridSpec(
        num_scalar_prefetch=4,
        grid=(NBLK,),
        in_specs=[pl.BlockSpec(memory_space=pl.ANY)],
        out_specs=pl.BlockSpec((BR, T), lambda i, *_: (i, 0)),
        scratch_shapes=[pltpu.SemaphoreType.DMA],
    )
    return pl.pallas_call(
        _body,
        grid_spec=grid_spec,
        out_shape=jax.ShapeDtypeStruct((B, T), jnp.float32),
    )(starts, ends, nf, oidx, e)


def kernel(e, att_starts, att_ends, n_att_frames, output_index):
    oidx = jnp.asarray(output_index, jnp.int32).reshape(1)
    return _tc_kernel(att_starts.astype(jnp.int32), att_ends.astype(jnp.int32),
                      n_att_frames.astype(jnp.int32), oidx, e)
